# baseline (speedup 1.0000x reference)
import functools

import jax
import jax.numpy as jnp
from jax import lax
from jax.experimental import pallas as pl
from jax.experimental.pallas import tpu as pltpu

IC = 128
NGROUP = 32
CPG = IC // NGROUP
EPSV = 1e-5
NSEG = 16


def _seg_mask(bid_row, rows):
    seg = lax.broadcasted_iota(jnp.int32, (NSEG, rows), 0)
    return bid_row[None, :] == seg


def _body(nblocks, rows, wr_ref, wa_ref, x_ref, bid_rd_ref, bid_wr_ref,
          w_ref, b_ref, o_ref, xs, s1, s2, cnt):
    t = pl.program_id(0)

    @pl.when(t == 0)
    def _():
        s1[...] = jnp.zeros_like(s1)
        s2[...] = jnp.zeros_like(s2)
        cnt[...] = jnp.zeros_like(cnt)

    @pl.when(t < nblocks)
    def _():
        x = x_ref[...]
        xs[pl.ds(t * rows, rows), :] = x
        mask = _seg_mask(bid_rd_ref[0, 0, :], rows)
        oht = mask.astype(jnp.bfloat16)
        xb = x.astype(jnp.bfloat16)
        s1[...] += lax.dot_general(oht, xb, (((1,), (0,)), ((), ())),
                                   preferred_element_type=jnp.float32)
        s2[...] += lax.dot_general(oht, xb * xb, (((1,), (0,)), ((), ())),
                                   preferred_element_type=jnp.float32)
        cnt[...] += jnp.sum(mask.astype(jnp.float32), axis=1)[:, None]

    @pl.when(wa_ref[t] == 1)
    def _():
        wb = wr_ref[t]
        ic = 1.0 / (cnt[...] * CPG + EPSV)
        ci = lax.broadcasted_iota(jnp.int32, (IC, IC), 0) // CPG
        cj = lax.broadcasted_iota(jnp.int32, (IC, IC), 1) // CPG
        ggt = (ci == cj).astype(jnp.float32)
        a1 = lax.dot_general(s1[...], ggt, (((1,), (0,)), ((), ())),
                             preferred_element_type=jnp.float32)
        a2 = lax.dot_general(s2[...], ggt, (((1,), (0,)), ((), ())),
                             preferred_element_type=jnp.float32)
        mg = a1 * ic
        var = ic * (a2 - 2.0 * mg * a1 + cnt[...] * CPG * mg * mg)
        istd = lax.rsqrt(var + EPSV)
        scale = istd * w_ref[...]
        shift = b_ref[...] - mg * scale
        tab = jnp.concatenate([scale, shift], axis=1).astype(jnp.bfloat16)

        x = xs[pl.ds(wb * rows, rows), :]
        oht = _seg_mask(bid_wr_ref[0, 0, :], rows).astype(jnp.bfloat16)
        rsh = lax.dot_general(oht, tab, (((0,), (0,)), ((), ())),
                              preferred_element_type=jnp.float32)
        o_ref[...] = x * rsh[:, :IC] + rsh[:, IC:]


def kernel(data, batch_id, batch_size, weights, bias):
    n, c = data.shape
    rows = 2000
    nblocks = n // rows
    assert nblocks * rows == n
    nstep = 2 * nblocks
    bid = batch_id.astype(jnp.int32)
    bid3 = bid.reshape(nblocks, 1, rows)

    seg_start = jnp.searchsorted(bid, jnp.arange(NSEG + 1),
                                 side="left").astype(jnp.int32)
    bid_last = bid[rows - 1::rows]
    ready_rows = seg_start[bid_last + 1]
    ready_blocks = (ready_rows + rows - 1) // rows
    idx = jnp.arange(nblocks, dtype=jnp.int32)
    wstep = lax.cummax(ready_blocks - 1 - idx) + idx
    tsteps = jnp.arange(nstep, dtype=jnp.int32)
    wr_map = jnp.clip(
        jnp.searchsorted(wstep, tsteps, side="right").astype(jnp.int32) - 1,
        0, nblocks - 1)
    w_active = jnp.zeros((nstep,), jnp.int32).at[wstep].set(1)

    grid_spec = pltpu.PrefetchScalarGridSpec(
        num_scalar_prefetch=2,
        grid=(nstep,),
        in_specs=[
            pl.BlockSpec((rows, c),
                         lambda t, wr, wa: (jnp.minimum(t, nblocks - 1), 0)),
            pl.BlockSpec((1, 1, rows),
                         lambda t, wr, wa: (jnp.minimum(t, nblocks - 1), 0, 0)),
            pl.BlockSpec((1, 1, rows), lambda t, wr, wa: (wr[t], 0, 0)),
            pl.BlockSpec((1, c), lambda t, wr, wa: (0, 0)),
            pl.BlockSpec((1, c), lambda t, wr, wa: (0, 0)),
        ],
        out_specs=pl.BlockSpec((rows, c), lambda t, wr, wa: (wr[t], 0)),
        scratch_shapes=[
            pltpu.VMEM((n, c), jnp.float32),
            pltpu.VMEM((NSEG, c), jnp.float32),
            pltpu.VMEM((NSEG, c), jnp.float32),
            pltpu.VMEM((NSEG, c), jnp.float32),
        ],
    )
    out = pl.pallas_call(
        functools.partial(_body, nblocks, rows),
        grid_spec=grid_spec,
        out_shape=jax.ShapeDtypeStruct((n, c), jnp.float32),
        compiler_params=pltpu.CompilerParams(
            dimension_semantics=("arbitrary",)),
    )(wr_map, w_active, data, bid3, bid3, weights, bias)
    return out
